# Initial kernel scaffold; baseline (speedup 1.0000x reference)
import jax
import jax.numpy as jnp
from jax.experimental import pallas as pl


def kernel(x, word_incices, table, fc_w, fc_b):
    raise NotImplementedError("write your pallas kernel here")


# speedup vs baseline: 3.0398x; 3.0398x over previous
import functools

import jax
import jax.numpy as jnp
from jax import lax
from jax.experimental import pallas as pl
from jax.experimental.pallas import tpu as pltpu
from jax.experimental.pallas import tpu_sc as plsc

_D = 64
_OUT = 100
_B = 4096
_L = 200
_W = 20

_NC = 2
_NS = 16
_NW = _NC * _NS
_BPW = _B // _NW
_CH = 100
_NCH = _L // _CH
_LANES = 16
_DV = _D // _LANES


def _sc_embed_sum(x3, table):
    mesh = plsc.VectorSubcoreMesh(
        core_axis_name="c", subcore_axis_name="s",
        num_cores=_NC, num_subcores=_NS)

    @functools.partial(
        pl.kernel,
        out_type=jax.ShapeDtypeStruct((_B, _D), jnp.float32),
        mesh=mesh,
        compiler_params=pltpu.CompilerParams(use_tc_tiling_on_sc=False),
        scratch_types=[
            pltpu.VMEM((_BPW, _NCH, _CH), jnp.int32),
            pltpu.VMEM((_NCH, _CH, _D), jnp.float32),
            pltpu.VMEM((_D,), jnp.float32),
            pltpu.SemaphoreType.DMA,
        ],
    )
    def body(x_hbm, tab_hbm, sent_hbm, idx_v, rows_v, acc_v, gsem):
        wid = lax.axis_index("s") * _NC + lax.axis_index("c")
        base = wid * _BPW
        pltpu.sync_copy(x_hbm.at[pl.ds(base, _BPW)], idx_v)

        @pl.loop(0, _BPW)
        def _row(i):
            for j in range(_NCH):
                pltpu.async_copy(
                    tab_hbm.at[idx_v.at[i, j]], rows_v.at[j], gsem).wait()

            zero = jnp.zeros((_LANES,), jnp.float32)
            acc = (zero,) * _DV

            def red(j):
                def f(c, carry):
                    return tuple(
                        carry[k] + rows_v[j, c, pl.ds(k * _LANES, _LANES)]
                        for k in range(_DV))
                return f

            for j in range(_NCH):
                acc = lax.fori_loop(0, _CH, red(j), acc)
            for k in range(_DV):
                acc_v[pl.ds(k * _LANES, _LANES)] = acc[k]
            pltpu.sync_copy(acc_v, sent_hbm.at[base + i])

    return body(x3, table)


def _fc_kernel(s_ref, w_ref, b_ref, o_ref):
    o_ref[...] = (
        jnp.dot(s_ref[...], w_ref[...], preferred_element_type=jnp.float32)
        * (1.0 / _W)
        + b_ref[...]
    )


def _fc(sent, w_t, fc_b):
    return pl.pallas_call(
        _fc_kernel,
        out_shape=jax.ShapeDtypeStruct((_B, _OUT), jnp.float32),
    )(sent, w_t, fc_b[None, :])


def kernel(x, word_incices, table, fc_w, fc_b):
    del word_incices
    x3 = x.reshape(_B, _NCH, _CH)
    sent = _sc_embed_sum(x3, table)
    return _fc(sent, fc_w.T, fc_b)


# speedup vs baseline: 3.8934x; 1.2808x over previous
import functools

import jax
import jax.numpy as jnp
from jax import lax
from jax.experimental import pallas as pl
from jax.experimental.pallas import tpu as pltpu
from jax.experimental.pallas import tpu_sc as plsc

_D = 64
_OUT = 100
_B = 4096
_L = 200
_W = 20

_NC = 2
_NS = 16
_NW = _NC * _NS
_BPW = _B // _NW
_CH = 100
_NCH = _L // _CH
_LANES = 16
_DV = _D // _LANES


def _sc_embed_sum(x3, table):
    mesh = plsc.VectorSubcoreMesh(
        core_axis_name="c", subcore_axis_name="s",
        num_cores=_NC, num_subcores=_NS)

    @functools.partial(
        pl.kernel,
        out_type=jax.ShapeDtypeStruct((_B, _D), jnp.float32),
        mesh=mesh,
        compiler_params=pltpu.CompilerParams(use_tc_tiling_on_sc=False),
        scratch_types=[
            pltpu.VMEM((_BPW, _NCH, _CH), jnp.int32),
            pltpu.VMEM((2, _NCH, _CH, _D), jnp.float32),
            pltpu.VMEM((_BPW, _D), jnp.float32),
            pltpu.SemaphoreType.DMA,
            pltpu.SemaphoreType.DMA,
        ],
    )
    def body(x_hbm, tab_hbm, sent_hbm, idx_v, rows_v, sums_v, gsem0, gsem1):
        wid = lax.axis_index("s") * _NC + lax.axis_index("c")
        base = wid * _BPW
        pltpu.sync_copy(x_hbm.at[pl.ds(base, _BPW)], idx_v)
        sems = (gsem0, gsem1)

        def fire(i, slot):
            for j in range(_NCH):
                pltpu.async_copy(
                    tab_hbm.at[idx_v.at[i, j]], rows_v.at[slot, j], sems[slot])

        def drain(slot):
            for j in range(_NCH):
                pltpu.make_async_copy(
                    tab_hbm.at[idx_v.at[0, j]], rows_v.at[slot, j],
                    sems[slot]).wait()

        fire(0, 0)
        fire(1, 1)

        @pl.loop(0, _BPW, step=2)
        def _rows(i):
            for b in range(2):
                ib = i + b
                drain(b)
                zero = jnp.zeros((_LANES,), jnp.float32)

                def red(j):
                    def f(c, carry):
                        return tuple(
                            carry[k] + rows_v[b, j, c, pl.ds(k * _LANES, _LANES)]
                            for k in range(_DV))
                    return f

                acc = (zero,) * _DV
                for j in range(_NCH):
                    acc = lax.fori_loop(0, _CH, red(j), acc, unroll=4)
                for k in range(_DV):
                    sums_v[ib, pl.ds(k * _LANES, _LANES)] = acc[k]

                @pl.when(ib + 2 < _BPW)
                def _():
                    fire(ib + 2, b)

        pltpu.sync_copy(sums_v, sent_hbm.at[pl.ds(base, _BPW)])

    return body(x3, table)


def _fc_kernel(s_ref, w_ref, b_ref, o_ref):
    o_ref[...] = (
        jnp.dot(s_ref[...], w_ref[...], preferred_element_type=jnp.float32)
        * (1.0 / _W)
        + b_ref[...]
    )


def _fc(sent, w_t, fc_b):
    return pl.pallas_call(
        _fc_kernel,
        out_shape=jax.ShapeDtypeStruct((_B, _OUT), jnp.float32),
    )(sent, w_t, fc_b[None, :])


def kernel(x, word_incices, table, fc_w, fc_b):
    del word_incices
    x3 = x.reshape(_B, _NCH, _CH)
    sent = _sc_embed_sum(x3, table)
    return _fc(sent, fc_w.T, fc_b)


# speedup vs baseline: 3.9150x; 1.0056x over previous
import functools

import jax
import jax.numpy as jnp
from jax import lax
from jax.experimental import pallas as pl
from jax.experimental.pallas import tpu as pltpu
from jax.experimental.pallas import tpu_sc as plsc

_D = 64
_OUT = 100
_B = 4096
_L = 200
_W = 20

_NC = 2
_NS = 16
_NW = _NC * _NS
_BPW = _B // _NW
_CHUNKS = ((0, 128), (128, 72))
_LANES = 16
_DV = _D // _LANES


def _sc_embed_sum(x, table):
    mesh = plsc.VectorSubcoreMesh(
        core_axis_name="c", subcore_axis_name="s",
        num_cores=_NC, num_subcores=_NS)

    @functools.partial(
        pl.kernel,
        out_type=jax.ShapeDtypeStruct((_B, _D), jnp.float32),
        mesh=mesh,
        compiler_params=pltpu.CompilerParams(use_tc_tiling_on_sc=False),
        scratch_types=[
            pltpu.VMEM((_BPW, _L), jnp.int32),
            pltpu.VMEM((2, _L, _D), jnp.float32),
            pltpu.VMEM((_BPW, _D), jnp.float32),
            pltpu.SemaphoreType.DMA,
            pltpu.SemaphoreType.DMA,
        ],
    )
    def body(x_hbm, tab_hbm, sent_hbm, idx_v, rows_v, sums_v, gsem0, gsem1):
        wid = lax.axis_index("s") * _NC + lax.axis_index("c")
        base = wid * _BPW
        pltpu.sync_copy(x_hbm.at[pl.ds(base, _BPW)], idx_v)
        sems = (gsem0, gsem1)

        def fire(i, slot):
            for off, sz in _CHUNKS:
                pltpu.async_copy(
                    tab_hbm.at[idx_v.at[i, pl.ds(off, sz)]],
                    rows_v.at[slot, pl.ds(off, sz)], sems[slot])

        def drain(slot):
            for off, sz in _CHUNKS:
                pltpu.make_async_copy(
                    tab_hbm.at[idx_v.at[0, pl.ds(off, sz)]],
                    rows_v.at[slot, pl.ds(off, sz)], sems[slot]).wait()

        fire(0, 0)
        fire(1, 1)

        @pl.loop(0, _BPW, step=2)
        def _rows(i):
            for b in range(2):
                ib = i + b
                drain(b)
                zero = jnp.zeros((_LANES,), jnp.float32)

                def red(c, carry):
                    return tuple(
                        carry[k] + rows_v[b, c, pl.ds(k * _LANES, _LANES)]
                        for k in range(_DV))

                acc = lax.fori_loop(0, _L, red, (zero,) * _DV, unroll=4)
                for k in range(_DV):
                    sums_v[ib, pl.ds(k * _LANES, _LANES)] = acc[k]

                @pl.when(ib + 2 < _BPW)
                def _():
                    fire(ib + 2, b)

        pltpu.sync_copy(sums_v, sent_hbm.at[pl.ds(base, _BPW)])

    return body(x, table)


def _fc_kernel(s_ref, w_ref, b_ref, o_ref):
    o_ref[...] = (
        jnp.dot(s_ref[...], w_ref[...], preferred_element_type=jnp.float32)
        * (1.0 / _W)
        + b_ref[...]
    )


def _fc(sent, w_t, fc_b):
    return pl.pallas_call(
        _fc_kernel,
        out_shape=jax.ShapeDtypeStruct((_B, _OUT), jnp.float32),
    )(sent, w_t, fc_b[None, :])


def kernel(x, word_incices, table, fc_w, fc_b):
    del word_incices
    sent = _sc_embed_sum(x, table)
    return _fc(sent, fc_w.T, fc_b)
